# Initial kernel scaffold; baseline (speedup 1.0000x reference)
import jax
import jax.numpy as jnp
from jax.experimental import pallas as pl


def kernel(token_words, trie_status, reward_mapping_values):
    raise NotImplementedError("write your pallas kernel here")


# speedup vs baseline: 129.7796x; 129.7796x over previous
import functools

import jax
import jax.numpy as jnp
from jax import lax
from jax.experimental import pallas as pl
from jax.experimental.pallas import tpu as pltpu
from jax.experimental.pallas import tpu_sc as plsc

END_TOKEN = 2
FULL_WORD_REWARD = 2.0

V = 1_000_000
TROWS, TCOLS = 1024, 1024
VPAD = TROWS * TCOLS
TBLK = 128

B, L = 16384, 200
N = B * L
NC, NS = 2, 16
NW = NC * NS
PER_W = N // NW
CHUNK = 10_240
NCHUNKS = PER_W // CHUNK


def _fuse_body(rm_ref, trie_ref, out_ref):
    s = trie_ref[...]
    idx = jnp.clip(s + 1, 0, 4)
    r = jnp.full(s.shape, rm_ref[0], dtype=jnp.float32)
    for k in range(1, 5):
        r = jnp.where(idx == k, rm_ref[k], r)
    gi = (pl.program_id(0) * TBLK + lax.broadcasted_iota(jnp.int32, s.shape, 0)) * TCOLS \
        + lax.broadcasted_iota(jnp.int32, s.shape, 1)
    out_ref[...] = jnp.where(gi == END_TOKEN, r + FULL_WORD_REWARD, r)


def _build_fused(trie_status, reward_mapping_values):
    t = jnp.pad(trie_status, (0, VPAD - V)).reshape(TROWS, TCOLS)
    fused = pl.pallas_call(
        _fuse_body,
        grid=(TROWS // TBLK,),
        in_specs=[
            pl.BlockSpec(memory_space=pltpu.SMEM),
            pl.BlockSpec((TBLK, TCOLS), lambda i: (i, 0)),
        ],
        out_specs=pl.BlockSpec((TBLK, TCOLS), lambda i: (i, 0)),
        out_shape=jax.ShapeDtypeStruct((TROWS, TCOLS), jnp.float32),
    )(reward_mapping_values, t)
    return fused.reshape(VPAD)


def _gather_body(tw_hbm, fused_hbm, out_hbm, idx_v, val_v, sem):
    wid = lax.axis_index("s") * NC + lax.axis_index("c")
    base = wid * PER_W

    def chunk(i, carry):
        r = base + i * CHUNK
        pltpu.sync_copy(tw_hbm.at[pl.ds(r, CHUNK)], idx_v)
        pltpu.async_copy(fused_hbm.at[idx_v], val_v, sem).wait()
        pltpu.sync_copy(val_v, out_hbm.at[pl.ds(r, CHUNK)])
        return carry

    lax.fori_loop(0, NCHUNKS, chunk, 0)


_gather_call = pl.kernel(
    _gather_body,
    mesh=plsc.VectorSubcoreMesh(core_axis_name="c", subcore_axis_name="s"),
    out_type=jax.ShapeDtypeStruct((N,), jnp.float32),
    scratch_types=[
        pltpu.VMEM((CHUNK,), jnp.int32),
        pltpu.VMEM((CHUNK,), jnp.float32),
        pltpu.SemaphoreType.DMA,
    ],
)


@jax.jit
def kernel(token_words, trie_status, reward_mapping_values):
    fused = _build_fused(trie_status, reward_mapping_values)
    out = _gather_call(token_words.reshape(N), fused)
    return out.reshape(B, L)


# speedup vs baseline: 223.7475x; 1.7241x over previous
import functools

import jax
import jax.numpy as jnp
from jax import lax
from jax.experimental import pallas as pl
from jax.experimental.pallas import tpu as pltpu
from jax.experimental.pallas import tpu_sc as plsc

END_TOKEN = 2
FULL_WORD_REWARD = 2.0

V = 1_000_000
TROWS, TCOLS = 1024, 1024
VPAD = TROWS * TCOLS
TBLK = 128

B, L = 16384, 200
N = B * L
NC, NS = 2, 16
NW = NC * NS
PER_W = N // NW
CHUNK = 10_240
NCHUNKS = PER_W // CHUNK


def _fuse_body(rm_ref, trie_ref, out_ref):
    s = trie_ref[...]
    idx = jnp.clip(s + 1, 0, 4)
    r = jnp.full(s.shape, rm_ref[0], dtype=jnp.float32)
    for k in range(1, 5):
        r = jnp.where(idx == k, rm_ref[k], r)
    gi = (pl.program_id(0) * TBLK + lax.broadcasted_iota(jnp.int32, s.shape, 0)) * TCOLS \
        + lax.broadcasted_iota(jnp.int32, s.shape, 1)
    out_ref[...] = jnp.where(gi == END_TOKEN, r + FULL_WORD_REWARD, r)


def _build_fused(trie_status, reward_mapping_values):
    t = jnp.pad(trie_status, (0, VPAD - V)).reshape(TROWS, TCOLS)
    fused = pl.pallas_call(
        _fuse_body,
        grid=(TROWS // TBLK,),
        in_specs=[
            pl.BlockSpec(memory_space=pltpu.SMEM),
            pl.BlockSpec((TBLK, TCOLS), lambda i: (i, 0)),
        ],
        out_specs=pl.BlockSpec((TBLK, TCOLS), lambda i: (i, 0)),
        out_shape=jax.ShapeDtypeStruct((TROWS, TCOLS), jnp.float32),
    )(reward_mapping_values, t)
    return fused.reshape(VPAD)


def _gather_body(tw_hbm, fused_hbm, out_hbm,
                 i0, i1, i2, v0, v1, fsh,
                 si0, si1, si2, sv0, sv1, so0, so1):
    cid = lax.axis_index("c")
    sid = lax.axis_index("s")
    wid = sid * NC + cid
    base = wid * PER_W
    idx = (i0, i1, i2)
    val = (v0, v1)
    isem = (si0, si1, si2)
    gsem = (sv0, sv1)
    osem = (so0, so1)

    ld = {}
    g = {}
    st = {}

    def start_ld(i):
        ld[i] = pltpu.async_copy(
            tw_hbm.at[pl.ds(base + i * CHUNK, CHUNK)], idx[i % 3], isem[i % 3])

    def start_st(i):
        st[i] = pltpu.async_copy(
            val[i % 2], out_hbm.at[pl.ds(base + i * CHUNK, CHUNK)], osem[i % 2])

    start_ld(0)
    start_ld(1)
    seg = VPAD // NS
    pltpu.sync_copy(fused_hbm.at[pl.ds(sid * seg, seg)],
                    fsh.at[pl.ds(sid * seg, seg)])
    plsc.subcore_barrier()

    for i in range(NCHUNKS):
        if i >= 2:
            st[i - 2].wait()
        ld[i].wait()
        g[i] = pltpu.async_copy(fsh.at[idx[i % 3]], val[i % 2], gsem[i % 2])
        if i >= 1:
            g[i - 1].wait()
            start_st(i - 1)
        if i + 2 < NCHUNKS:
            start_ld(i + 2)
    g[NCHUNKS - 1].wait()
    start_st(NCHUNKS - 1)
    st[NCHUNKS - 2].wait()
    st[NCHUNKS - 1].wait()


_gather_call = pl.kernel(
    _gather_body,
    mesh=plsc.VectorSubcoreMesh(core_axis_name="c", subcore_axis_name="s"),
    out_type=jax.ShapeDtypeStruct((N,), jnp.float32),
    scratch_types=[
        pltpu.VMEM((CHUNK,), jnp.int32),
        pltpu.VMEM((CHUNK,), jnp.int32),
        pltpu.VMEM((CHUNK,), jnp.int32),
        pltpu.VMEM((CHUNK,), jnp.float32),
        pltpu.VMEM((CHUNK,), jnp.float32),
        pltpu.VMEM_SHARED((VPAD,), jnp.float32),
        pltpu.SemaphoreType.DMA,
        pltpu.SemaphoreType.DMA,
        pltpu.SemaphoreType.DMA,
        pltpu.SemaphoreType.DMA,
        pltpu.SemaphoreType.DMA,
        pltpu.SemaphoreType.DMA,
        pltpu.SemaphoreType.DMA,
    ],
)


@jax.jit
def kernel(token_words, trie_status, reward_mapping_values):
    fused = _build_fused(trie_status, reward_mapping_values)
    out = _gather_call(token_words.reshape(N), fused)
    return out.reshape(B, L)
